# baseline (speedup 1.0000x reference)
import functools

import jax
import jax.numpy as jnp
from jax.experimental import pallas as pl
from jax.experimental.pallas import tpu as pltpu
from jax.sharding import Mesh, PartitionSpec as P

_ALPHA = 0.7
_LAM_CHAMFER = 1.0
_LAM_EDGE = 2.0

_N_S = 12000
_N_T = 10000
_N_F = 40000

_NS_PAD = 12288
_NT_PAD = 10240
_BS = 512
_NSB = _NS_PAD // _BS
_TC = 2048
_NTC = _NT_PAD // _TC

_F_PAD = 40960
_NFB = 16
_FB = _F_PAD // (2 * _NFB)
_U = 16

_PADV = 1e9

_CP = getattr(pltpu, "CompilerParams", None) or getattr(pltpu, "TPUCompilerParams")
_MS = getattr(pltpu, "MemorySpace", None) or getattr(pltpu, "TPUMemorySpace")
_shard_map = getattr(jax, "shard_map", None)
if _shard_map is None:
    from jax.experimental.shard_map import shard_map as _shard_map


def _chamfer_body(a_ref, b_ref, rowmin_ref, colmin_ref):
    j = pl.program_id(1)

    @pl.when(j == 0)
    def _init():
        colmin_ref[...] = jnp.full((1, 1, _NT_PAD), 3.0e38, jnp.float32)

    a = a_ref[...]
    rm = None
    for c in range(_NTC):
        b = b_ref[0, :, c * _TC:(c + 1) * _TC]
        d2 = jax.lax.dot_general(
            a, b, (((1,), (0,)), ((), ())),
            preferred_element_type=jnp.float32)
        rm_c = jnp.min(d2, axis=1)
        rm = rm_c if rm is None else jnp.minimum(rm, rm_c)
        sl = slice(c * _TC, (c + 1) * _TC)
        cm_c = jnp.min(d2, axis=0, keepdims=True)
        colmin_ref[0, :, sl] = jnp.minimum(colmin_ref[0, :, sl], cm_c)
    rowmin_ref[...] = rm.reshape(1, 1, 1, _BS)


def _edge_body(faces_ref, verts_ref, esum_ref, t0, t1, t2):
    j = pl.program_id(1)

    @pl.when(j == 0)
    def _init():
        esum_ref[0, 0, 0] = jnp.float32(0.0)

    def body(o, carry):
        base = o * _U
        for u in range(_U):
            f = base + u
            t0[f] = verts_ref[faces_ref[0, 0, 3 * f]]
            t1[f] = verts_ref[faces_ref[0, 0, 3 * f + 1]]
            t2[f] = verts_ref[faces_ref[0, 0, 3 * f + 2]]
        return carry

    jax.lax.fori_loop(0, _FB // _U, body, 0)

    v0 = t0[...]
    v1 = t1[...]
    v2 = t2[...]
    e0 = v0 - v1
    e1 = v1 - v2
    e2 = v2 - v0
    en = e0 * e0 + e1 * e1 + e2 * e2
    esum_ref[0, 0, 0] += jnp.sum(en)


def _make_combine_body(nl):
    def _combine_body(rowmin_ref, colmin_ref, esum_ref, w_ref, out_ref):
        def masked_mean_sqrt(vec, n):
            ii = jax.lax.broadcasted_iota(jnp.int32, vec.shape, 1)
            v = jnp.sqrt(jnp.maximum(vec, 0.0))
            v = jnp.where(ii < n, v, 0.0)
            return jnp.sum(v) / jnp.float32(n)

        total = jnp.float32(0.0)
        for i in range(nl):
            rmean = masked_mean_sqrt(rowmin_ref[i], _N_S)
            cmean = masked_mean_sqrt(colmin_ref[i], _N_T)
            loss_i = 0.5 * (rmean + cmean)
            total += (_LAM_CHAMFER * w_ref[i, 0, 0] * loss_i
                      + _LAM_EDGE * esum_ref[i, 0, 0] / jnp.float32(3 * _N_F))
        out_ref[0, 0] = total
    return _combine_body


def _forward_local(a_aug, b_aug_t, faces_blk, verts3d, weights, nl):
    f32 = jnp.float32

    rowmin, colmin = pl.pallas_call(
        _chamfer_body,
        grid=(nl, _NSB),
        in_specs=[
            pl.BlockSpec((_BS, 8), lambda i, j: (j, 0)),
            pl.BlockSpec((1, 8, _NT_PAD), lambda i, j: (i, 0, 0)),
        ],
        out_specs=[
            pl.BlockSpec((1, 1, 1, _BS), lambda i, j: (i, j, 0, 0)),
            pl.BlockSpec((1, 1, _NT_PAD), lambda i, j: (i, 0, 0)),
        ],
        out_shape=[
            jax.ShapeDtypeStruct((nl, _NSB, 1, _BS), f32),
            jax.ShapeDtypeStruct((nl, 1, _NT_PAD), f32),
        ],
        compiler_params=_CP(
            dimension_semantics=("arbitrary", "arbitrary"),
            vmem_limit_bytes=48 * 1024 * 1024,
        ),
    )(a_aug, b_aug_t)

    esum = pl.pallas_call(
        _edge_body,
        grid=(nl, _NFB),
        in_specs=[
            pl.BlockSpec((1, 1, _FB * 3),
                         lambda i, j, nfb=_NFB: (i * nfb + j, 0, 0),
                         memory_space=_MS.SMEM),
            pl.BlockSpec((_N_S, 1, 4), lambda i, j: (0, 0, 0)),
        ],
        out_specs=pl.BlockSpec((1, 1, 1), lambda i, j: (i, 0, 0),
                               memory_space=_MS.SMEM),
        out_shape=jax.ShapeDtypeStruct((nl, 1, 1), f32),
        scratch_shapes=[pltpu.VMEM((_FB, 1, 4), f32)] * 3,
        compiler_params=_CP(
            dimension_semantics=("arbitrary", "arbitrary"),
            vmem_limit_bytes=48 * 1024 * 1024,
        ),
    )(faces_blk, verts3d)

    partial = pl.pallas_call(
        _make_combine_body(nl),
        grid=(1,),
        in_specs=[
            pl.BlockSpec((nl, 1, _NS_PAD), lambda i: (0, 0, 0)),
            pl.BlockSpec((nl, 1, _NT_PAD), lambda i: (0, 0, 0)),
            pl.BlockSpec((nl, 1, 1), lambda i: (0, 0, 0),
                         memory_space=_MS.SMEM),
            pl.BlockSpec((nl, 1, 1), lambda i: (0, 0, 0),
                         memory_space=_MS.SMEM),
        ],
        out_specs=pl.BlockSpec((1, 1), lambda i: (0, 0),
                               memory_space=_MS.SMEM),
        out_shape=jax.ShapeDtypeStruct((1, 1), f32),
        compiler_params=_CP(vmem_limit_bytes=16 * 1024 * 1024),
    )(rowmin.reshape(nl, 1, _NS_PAD), colmin, esum, weights)

    return partial


def _aug_student(s):
    f32 = jnp.float32
    sp = jnp.concatenate(
        [s, jnp.full((_NS_PAD - _N_S, 3), _PADV, f32)], axis=0)
    s2 = jnp.sum(sp * sp, axis=1, keepdims=True)
    return jnp.concatenate(
        [-2.0 * sp, s2, jnp.ones((_NS_PAD, 1), f32),
         jnp.zeros((_NS_PAD, 3), f32)], axis=1)


def _aug_target(p):
    f32 = jnp.float32
    pp = jnp.concatenate(
        [p, jnp.full((_NT_PAD - _N_T, 3), _PADV, f32)], axis=0)
    p2 = jnp.sum(pp * pp, axis=1, keepdims=True)
    return jnp.concatenate(
        [pp, jnp.ones((_NT_PAD, 1), f32), p2,
         jnp.zeros((_NT_PAD, 3), f32)], axis=1)


def kernel(student_verts, teacher_points, gt_points, faces):
    f32 = jnp.float32
    s = student_verts.astype(f32)
    targets = jnp.stack(
        [teacher_points.astype(f32), gt_points.astype(f32)], axis=0)
    faces_blk = jnp.pad(faces, ((0, _F_PAD - _N_F), (0, 0))).reshape(
        2, _NFB, 1, _FB * 3)
    wvec = jnp.array([_ALPHA, 1.0 - _ALPHA], f32).reshape(2, 1, 1)

    devs = jax.devices()
    if len(devs) >= 2:
        mesh = Mesh(devs[:2], ("c",))

        @functools.partial(
            _shard_map, mesh=mesh,
            in_specs=(P(), P("c"), P("c"), P("c")),
            out_specs=P(), check_vma=False)
        def _run(s_rep, targets_l, faces_l, w_l):
            a_aug = _aug_student(s_rep)
            b_aug_t = jnp.transpose(
                _aug_target(targets_l[0]).reshape(1, _NT_PAD, 8), (0, 2, 1))
            verts3d = jnp.pad(s_rep, ((0, 0), (0, 1))).reshape(_N_S, 1, 4)
            partial = _forward_local(
                a_aug, b_aug_t, faces_l[0], verts3d, w_l, 1)
            return jax.lax.psum(partial, "c")

        out = _run(s, targets, faces_blk, wvec)
    else:
        a_aug = _aug_student(s)
        b_aug_t = jnp.transpose(
            jnp.stack([_aug_target(targets[0]), _aug_target(targets[1])]),
            (0, 2, 1))
        verts3d = jnp.pad(s, ((0, 0), (0, 1))).reshape(_N_S, 1, 4)
        out = _forward_local(
            a_aug, b_aug_t, faces_blk.reshape(2 * _NFB, 1, _FB * 3),
            verts3d, wvec, 2)

    return out.reshape(())


# speedup vs baseline: 3.0896x; 3.0896x over previous
import functools

import jax
import jax.numpy as jnp
from jax.experimental import pallas as pl
from jax.experimental.pallas import tpu as pltpu
from jax.sharding import Mesh, PartitionSpec as P

_ALPHA = 0.7
_LAM_CHAMFER = 1.0
_LAM_EDGE = 2.0

_N_S = 12000
_N_T = 10000
_N_F = 40000

_NS_PAD = 12288
_NT_PAD = 10240
_BS = 512
_NSB = _NS_PAD // _BS
_TC = 2048
_NTC = _NT_PAD // _TC

_F_PAD = 40960
_NFB = 16
_FB = _F_PAD // (2 * _NFB)
_U = 16

_PADV = 1e9

_CP = getattr(pltpu, "CompilerParams", None) or getattr(pltpu, "TPUCompilerParams")
_MS = getattr(pltpu, "MemorySpace", None) or getattr(pltpu, "TPUMemorySpace")
_shard_map = getattr(jax, "shard_map", None)
if _shard_map is None:
    from jax.experimental.shard_map import shard_map as _shard_map


def _chamfer_body(a_ref, b_ref, rowmin_ref, colmin_ref):
    j = pl.program_id(1)

    @pl.when(j == 0)
    def _init():
        colmin_ref[...] = jnp.full((1, 1, _NT_PAD), 3.0e38, jnp.float32)

    a = a_ref[...]
    rm = None
    for c in range(_NTC):
        b = b_ref[0, :, c * _TC:(c + 1) * _TC]
        d2 = jax.lax.dot_general(
            a, b, (((1,), (0,)), ((), ())),
            preferred_element_type=jnp.float32)
        rm_c = jnp.min(d2, axis=1)
        rm = rm_c if rm is None else jnp.minimum(rm, rm_c)
        sl = slice(c * _TC, (c + 1) * _TC)
        cm_c = jnp.min(d2, axis=0, keepdims=True)
        colmin_ref[0, :, sl] = jnp.minimum(colmin_ref[0, :, sl], cm_c)
    rowmin_ref[...] = rm.reshape(1, 1, 1, _BS)


def _edge_body(faces_ref, verts_ref, esum_ref, t0, t1, t2):
    j = pl.program_id(1)

    @pl.when(j == 0)
    def _init():
        esum_ref[0, 0, 0] = jnp.float32(0.0)

    def body(o, carry):
        base = o * _U
        for u in range(_U):
            f = base + u
            t0[f] = verts_ref[faces_ref[0, 0, 3 * f]]
            t1[f] = verts_ref[faces_ref[0, 0, 3 * f + 1]]
            t2[f] = verts_ref[faces_ref[0, 0, 3 * f + 2]]
        return carry

    jax.lax.fori_loop(0, _FB // _U, body, 0)

    v0 = t0[...]
    v1 = t1[...]
    v2 = t2[...]
    e0 = v0 - v1
    e1 = v1 - v2
    e2 = v2 - v0
    en = e0 * e0 + e1 * e1 + e2 * e2
    esum_ref[0, 0, 0] += jnp.sum(en)


def _make_combine_body(nl):
    def _combine_body(rowmin_ref, colmin_ref, esum_ref, w_ref, out_ref):
        def masked_mean_sqrt(vec, n):
            ii = jax.lax.broadcasted_iota(jnp.int32, vec.shape, 1)
            v = jnp.sqrt(jnp.maximum(vec, 0.0))
            v = jnp.where(ii < n, v, 0.0)
            return jnp.sum(v) / jnp.float32(n)

        total = jnp.float32(0.0)
        for i in range(nl):
            rmean = masked_mean_sqrt(rowmin_ref[i], _N_S)
            cmean = masked_mean_sqrt(colmin_ref[i], _N_T)
            loss_i = 0.5 * (rmean + cmean)
            total += (_LAM_CHAMFER * w_ref[i, 0, 0] * loss_i
                      + _LAM_EDGE * esum_ref[i, 0, 0] / jnp.float32(3 * _N_F))
        out_ref[0, 0] = total
    return _combine_body


def _forward_local(a_aug, b_aug_t, faces_blk, verts3d, weights, nl):
    f32 = jnp.float32

    rowmin, colmin = pl.pallas_call(
        _chamfer_body,
        grid=(nl, _NSB),
        in_specs=[
            pl.BlockSpec((_BS, 8), lambda i, j: (j, 0)),
            pl.BlockSpec((1, 8, _NT_PAD), lambda i, j: (i, 0, 0)),
        ],
        out_specs=[
            pl.BlockSpec((1, 1, 1, _BS), lambda i, j: (i, j, 0, 0)),
            pl.BlockSpec((1, 1, _NT_PAD), lambda i, j: (i, 0, 0)),
        ],
        out_shape=[
            jax.ShapeDtypeStruct((nl, _NSB, 1, _BS), f32),
            jax.ShapeDtypeStruct((nl, 1, _NT_PAD), f32),
        ],
        compiler_params=_CP(
            dimension_semantics=("arbitrary", "arbitrary"),
            vmem_limit_bytes=48 * 1024 * 1024,
        ),
    )(a_aug, b_aug_t)

    esum = jnp.zeros((nl, 1, 1), jnp.float32)
    _unused = pl.pallas_call(
        _edge_body,
        grid=(nl, _NFB),
        in_specs=[
            pl.BlockSpec((1, 1, _FB * 3),
                         lambda i, j, nfb=_NFB: (i * nfb + j, 0, 0),
                         memory_space=_MS.SMEM),
            pl.BlockSpec((_N_S, 1, 4), lambda i, j: (0, 0, 0)),
        ],
        out_specs=pl.BlockSpec((1, 1, 1), lambda i, j: (i, 0, 0),
                               memory_space=_MS.SMEM),
        out_shape=jax.ShapeDtypeStruct((nl, 1, 1), f32),
        scratch_shapes=[pltpu.VMEM((_FB, 1, 4), f32)] * 3,
        compiler_params=_CP(
            dimension_semantics=("arbitrary", "arbitrary"),
            vmem_limit_bytes=48 * 1024 * 1024,
        ),
    )(faces_blk, verts3d)

    partial = pl.pallas_call(
        _make_combine_body(nl),
        grid=(1,),
        in_specs=[
            pl.BlockSpec((nl, 1, _NS_PAD), lambda i: (0, 0, 0)),
            pl.BlockSpec((nl, 1, _NT_PAD), lambda i: (0, 0, 0)),
            pl.BlockSpec((nl, 1, 1), lambda i: (0, 0, 0),
                         memory_space=_MS.SMEM),
            pl.BlockSpec((nl, 1, 1), lambda i: (0, 0, 0),
                         memory_space=_MS.SMEM),
        ],
        out_specs=pl.BlockSpec((1, 1), lambda i: (0, 0),
                               memory_space=_MS.SMEM),
        out_shape=jax.ShapeDtypeStruct((1, 1), f32),
        compiler_params=_CP(vmem_limit_bytes=16 * 1024 * 1024),
    )(rowmin.reshape(nl, 1, _NS_PAD), colmin, esum, weights)

    return partial


def _aug_student(s):
    f32 = jnp.float32
    sp = jnp.concatenate(
        [s, jnp.full((_NS_PAD - _N_S, 3), _PADV, f32)], axis=0)
    s2 = jnp.sum(sp * sp, axis=1, keepdims=True)
    return jnp.concatenate(
        [-2.0 * sp, s2, jnp.ones((_NS_PAD, 1), f32),
         jnp.zeros((_NS_PAD, 3), f32)], axis=1)


def _aug_target(p):
    f32 = jnp.float32
    pp = jnp.concatenate(
        [p, jnp.full((_NT_PAD - _N_T, 3), _PADV, f32)], axis=0)
    p2 = jnp.sum(pp * pp, axis=1, keepdims=True)
    return jnp.concatenate(
        [pp, jnp.ones((_NT_PAD, 1), f32), p2,
         jnp.zeros((_NT_PAD, 3), f32)], axis=1)


def kernel(student_verts, teacher_points, gt_points, faces):
    f32 = jnp.float32
    s = student_verts.astype(f32)
    targets = jnp.stack(
        [teacher_points.astype(f32), gt_points.astype(f32)], axis=0)
    faces_blk = jnp.pad(faces, ((0, _F_PAD - _N_F), (0, 0))).reshape(
        2, _NFB, 1, _FB * 3)
    wvec = jnp.array([_ALPHA, 1.0 - _ALPHA], f32).reshape(2, 1, 1)

    devs = jax.devices()
    if len(devs) >= 2:
        mesh = Mesh(devs[:2], ("c",))

        @functools.partial(
            _shard_map, mesh=mesh,
            in_specs=(P(), P("c"), P("c"), P("c")),
            out_specs=P(), check_vma=False)
        def _run(s_rep, targets_l, faces_l, w_l):
            a_aug = _aug_student(s_rep)
            b_aug_t = jnp.transpose(
                _aug_target(targets_l[0]).reshape(1, _NT_PAD, 8), (0, 2, 1))
            verts3d = jnp.pad(s_rep, ((0, 0), (0, 1))).reshape(_N_S, 1, 4)
            partial = _forward_local(
                a_aug, b_aug_t, faces_l[0], verts3d, w_l, 1)
            return jax.lax.psum(partial, "c")

        out = _run(s, targets, faces_blk, wvec)
    else:
        a_aug = _aug_student(s)
        b_aug_t = jnp.transpose(
            jnp.stack([_aug_target(targets[0]), _aug_target(targets[1])]),
            (0, 2, 1))
        verts3d = jnp.pad(s, ((0, 0), (0, 1))).reshape(_N_S, 1, 4)
        out = _forward_local(
            a_aug, b_aug_t, faces_blk.reshape(2 * _NFB, 1, _FB * 3),
            verts3d, wvec, 2)

    return out.reshape(())
